# baseline (speedup 1.0000x reference)
import functools

import jax
import jax.numpy as jnp
from jax import lax
from jax.experimental import pallas as pl
from jax.experimental.pallas import tpu as pltpu
from jax.experimental.pallas import tpu_sc as plsc

N_NODES = 10000
N_EDGES = 320000
D = 128

NC = 2
NS = 16
L = 16
NW = NC * NS
E_PER_W = N_EDGES // NW
CHUNK = 80
N_CHUNKS = E_PER_W // CHUNK
NBUF = 3
DW_CHUNKS = 63
DW_HALF = DW_CHUNKS * CHUNK
ROWS_PER_TILE = 624
TAIL_ROWS = N_NODES - ROWS_PER_TILE * NS


def _weight_mul(dwloc, lc, rows_p, didx_p):
    base = pl.multiple_of(lc * CHUNK, 8)

    def group_body(g, _):
        off = pl.multiple_of(base + g * L, 8)
        dw = dwloc[pl.ds(off, L)]
        didx_p[pl.ds(g * L, L)] = dw & jnp.int32(0xFFFF)
        wv = plsc.bitcast(dw & jnp.int32(-65536), jnp.float32)
        for i in range(L):
            ws = jnp.full((L,), wv[i], jnp.float32)
            e = g * L + i
            for j in range(D // L):
                sl = pl.ds(j * L, L)
                rows_p[e, sl] = rows_p[e, sl] * ws
        return 0
    lax.fori_loop(0, CHUNK // L, group_body, 0)


def _spmm_body(x_hbm, src_hbm, dw_hbm, out_hbm,
               sloc, dwloc, rows0, rows1, rows2, didx0, didx1, didx2,
               acc, rsem, gsem0, gsem1, gsem2, ssem0, ssem1, ssem2):
    cid = lax.axis_index("c")
    sid = lax.axis_index("s")
    wid = cid * NS + sid
    base_e = wid * E_PER_W

    esl = pl.ds(base_e, E_PER_W)
    dsl = pl.ds(base_e, DW_HALF)
    pltpu.make_async_copy(src_hbm.at[esl], sloc, rsem).start()
    pltpu.make_async_copy(dw_hbm.at[dsl], dwloc, rsem).start()

    def zero_row(i, _):
        for j in range(D // L):
            rows0[i, pl.ds(j * L, L)] = jnp.zeros((L,), jnp.float32)
        return 0
    lax.fori_loop(0, CHUNK, zero_row, 0)
    slab0 = sid * ROWS_PER_TILE

    def zero_copy(k, _):
        off = pl.multiple_of(slab0 + k * CHUNK, 8)
        pltpu.sync_copy(rows0, acc.at[pl.ds(off, CHUNK)])
        return 0
    n_full = ROWS_PER_TILE // CHUNK
    z_tail = ROWS_PER_TILE - n_full * CHUNK
    lax.fori_loop(0, n_full, zero_copy, 0)
    pltpu.sync_copy(rows0.at[pl.ds(0, z_tail)],
                    acc.at[pl.ds(slab0 + n_full * CHUNK, z_tail)])

    @pl.when(sid == NS - 1)
    def _zero_tail():
        pltpu.sync_copy(rows0.at[pl.ds(0, TAIL_ROWS)],
                        acc.at[pl.ds(NS * ROWS_PER_TILE, TAIL_ROWS)])
    plsc.subcore_barrier()

    rows = (rows0, rows1, rows2)
    didx = (didx0, didx1, didx2)
    gsem = (gsem0, gsem1, gsem2)
    ssem = (ssem0, ssem1, ssem2)

    def src_idx(c):
        return sloc.at[pl.ds(pl.multiple_of(c * CHUNK, 8), CHUNK)]

    def gather(c, j):
        return pltpu.make_async_copy(x_hbm.at[src_idx(c)], rows[j], gsem[j])

    def scatter_start(j):
        pltpu.async_copy(rows[j], acc.at[didx[j]], ssem[j], add=True)

    def scatter_wait(j):
        pltpu.make_async_copy(rows[j], acc.at[didx[j]], ssem[j]).wait()

    pltpu.make_async_copy(src_hbm.at[esl], sloc, rsem).wait()
    pltpu.make_async_copy(dw_hbm.at[dsl], dwloc, rsem).wait()
    gather(0, 0).start()

    def pipe_body(k, _):
        for j in range(NBUF):
            c = NBUF * k + j
            nj = (j + 1) % NBUF
            gather(c, j).wait()
            if j < 2:
                @pl.when(k >= 1)
                def _():
                    scatter_wait(nj)
            else:
                scatter_wait(nj)
            gather(c + 1, nj).start()
            lc = jnp.where(k >= DW_CHUNKS // NBUF, c - DW_CHUNKS, c)
            _weight_mul(dwloc, lc, rows[j], didx[j])
            scatter_start(j)

        @pl.when(k == DW_CHUNKS // NBUF - 1)
        def _():
            pltpu.sync_copy(
                dw_hbm.at[pl.ds(base_e + DW_HALF, E_PER_W - DW_HALF)],
                dwloc.at[pl.ds(0, E_PER_W - DW_HALF)])
        return 0

    lax.fori_loop(0, (N_CHUNKS - 2) // NBUF, pipe_body, 0)

    c = N_CHUNKS - 2
    gather(c, 0).wait()
    scatter_wait(1)
    gather(c + 1, 1).start()
    _weight_mul(dwloc, c - DW_CHUNKS, rows0, didx0)
    scatter_start(0)

    gather(c + 1, 1).wait()
    _weight_mul(dwloc, c + 1 - DW_CHUNKS, rows1, didx1)
    scatter_start(1)

    scatter_wait(2)
    scatter_wait(0)
    scatter_wait(1)
    plsc.subcore_barrier()

    sl = pl.ds(slab0, ROWS_PER_TILE)
    pltpu.sync_copy(acc.at[sl], out_hbm.at[cid, sl])

    @pl.when(sid == NS - 1)
    def _flush_tail():
        tl = pl.ds(NS * ROWS_PER_TILE, TAIL_ROWS)
        pltpu.sync_copy(acc.at[tl], out_hbm.at[cid, tl])


_spmm = functools.partial(
    pl.kernel,
    out_type=jax.ShapeDtypeStruct((NC, N_NODES, D), jnp.float32),
    mesh=plsc.VectorSubcoreMesh(core_axis_name="c", subcore_axis_name="s"),
    compiler_params=pltpu.CompilerParams(needs_layout_passes=False),
    scratch_types=[
        pltpu.VMEM((E_PER_W,), jnp.int32),
        pltpu.VMEM((DW_HALF,), jnp.int32),
        pltpu.VMEM((CHUNK, D), jnp.float32),
        pltpu.VMEM((CHUNK, D), jnp.float32),
        pltpu.VMEM((CHUNK, D), jnp.float32),
        pltpu.VMEM((CHUNK,), jnp.int32),
        pltpu.VMEM((CHUNK,), jnp.int32),
        pltpu.VMEM((CHUNK,), jnp.int32),
        pltpu.VMEM_SHARED((N_NODES, D), jnp.float32),
        pltpu.SemaphoreType.DMA,
        pltpu.SemaphoreType.DMA,
        pltpu.SemaphoreType.DMA,
        pltpu.SemaphoreType.DMA,
        pltpu.SemaphoreType.DMA,
        pltpu.SemaphoreType.DMA,
        pltpu.SemaphoreType.DMA,
    ],
)(_spmm_body)


def _mm_body(p_ref, w_ref, b_ref, o_ref):
    s = p_ref[0] + p_ref[1]
    o_ref[...] = (
        jnp.dot(s, w_ref[...], preferred_element_type=jnp.float32)
        + b_ref[...]
    )


M_BLK = 1000


def _fused_matmul(partials, W, b):
    return pl.pallas_call(
        _mm_body,
        grid=(N_NODES // M_BLK,),
        in_specs=[
            pl.BlockSpec((NC, M_BLK, D), lambda i: (0, i, 0)),
            pl.BlockSpec((D, D), lambda i: (0, 0)),
            pl.BlockSpec((1, D), lambda i: (0, 0)),
        ],
        out_specs=pl.BlockSpec((M_BLK, D), lambda i: (i, 0)),
        out_shape=jax.ShapeDtypeStruct((N_NODES, D), jnp.float32),
    )(partials, W, b.reshape(1, D))


def kernel(x, edge_index, edge_weight, W, b):
    ei = edge_index.astype(jnp.int32)
    wbits = lax.bitcast_convert_type(
        edge_weight.astype(jnp.bfloat16), jnp.uint16).astype(jnp.int32)
    dw = (wbits << 16) | ei[0]
    partials = _spmm(x, ei[1], dw)
    return _fused_matmul(partials, W, b)


# speedup vs baseline: 1.1303x; 1.1303x over previous
import functools

import jax
import jax.numpy as jnp
from jax import lax
from jax.experimental import pallas as pl
from jax.experimental.pallas import tpu as pltpu
from jax.experimental.pallas import tpu_sc as plsc

N_NODES = 10000
N_EDGES = 320000
D = 128

NC = 2
NS = 16
L = 16
NW = NC * NS
E_PER_W = N_EDGES // NW
CHUNK = 128
N_FULL = E_PER_W // CHUNK
TAIL_E = E_PER_W - N_FULL * CHUNK
DW_CHUNKS = 39
DW_HALF = DW_CHUNKS * CHUNK
DW_REST = E_PER_W - DW_HALF
ROWS_PER_TILE = 624
TAIL_ROWS = N_NODES - ROWS_PER_TILE * NS


def _weight_mul(dwloc, lbase, rows_p, didx_p, n_groups):
    def group_body(g, _):
        off = pl.multiple_of(lbase + g * L, 8)
        dw = dwloc[pl.ds(off, L)]
        didx_p[pl.ds(g * L, L)] = dw & jnp.int32(0xFFFF)
        wv = plsc.bitcast(dw & jnp.int32(-65536), jnp.float32)
        for i in range(L):
            ws = jnp.full((L,), wv[i], jnp.float32)
            e = g * L + i
            for j in range(D // L):
                sl = pl.ds(j * L, L)
                rows_p[e, sl] = rows_p[e, sl] * ws
        return 0
    lax.fori_loop(0, n_groups, group_body, 0)


def _spmm_body(x_hbm, src_hbm, dw_hbm, out_hbm,
               sloc, dwloc, rows0, rows1, trows, didx0, didx1, dtail,
               acc, rsem, gsem0, gsem1, ssem0, ssem1):
    cid = lax.axis_index("c")
    sid = lax.axis_index("s")
    wid = cid * NS + sid
    base_e = wid * E_PER_W

    esl = pl.ds(base_e, E_PER_W)
    dsl = pl.ds(base_e, DW_HALF)
    pltpu.make_async_copy(src_hbm.at[esl], sloc, rsem).start()
    pltpu.make_async_copy(dw_hbm.at[dsl], dwloc.at[pl.ds(0, DW_HALF)], rsem).start()

    def zero_row(i, _):
        for j in range(D // L):
            rows0[i, pl.ds(j * L, L)] = jnp.zeros((L,), jnp.float32)
        return 0
    lax.fori_loop(0, CHUNK, zero_row, 0)
    slab0 = sid * ROWS_PER_TILE

    def zero_copy(k, _):
        off = pl.multiple_of(slab0 + k * CHUNK, 8)
        pltpu.sync_copy(rows0, acc.at[pl.ds(off, CHUNK)])
        return 0
    n_zfull = ROWS_PER_TILE // CHUNK
    z_tail = ROWS_PER_TILE - n_zfull * CHUNK
    lax.fori_loop(0, n_zfull, zero_copy, 0)
    pltpu.sync_copy(rows0.at[pl.ds(0, z_tail)],
                    acc.at[pl.ds(slab0 + n_zfull * CHUNK, z_tail)])

    @pl.when(sid == NS - 1)
    def _zero_tail():
        pltpu.sync_copy(rows0.at[pl.ds(0, TAIL_ROWS)],
                        acc.at[pl.ds(NS * ROWS_PER_TILE, TAIL_ROWS)])
    plsc.subcore_barrier()

    def src_idx(c):
        return sloc.at[pl.ds(pl.multiple_of(c * CHUNK, 8), CHUNK)]

    def gather(c, rows, gsem):
        return pltpu.make_async_copy(x_hbm.at[src_idx(c)], rows, gsem)

    def scatter_start(rows, didx, ssem):
        pltpu.async_copy(rows, acc.at[didx], ssem, add=True)

    def scatter_wait(rows, didx, ssem):
        pltpu.make_async_copy(rows, acc.at[didx], ssem).wait()

    pltpu.make_async_copy(src_hbm.at[esl], sloc, rsem).wait()
    pltpu.make_async_copy(dw_hbm.at[dsl], dwloc.at[pl.ds(0, DW_HALF)], rsem).wait()
    gather(0, rows0, gsem0).start()

    def pipe_body(k, _):
        c0 = 2 * k
        c1 = 2 * k + 1
        gather(c0, rows0, gsem0).wait()

        @pl.when(k >= 1)
        def _():
            scatter_wait(rows1, didx1, ssem1)
        gather(c1, rows1, gsem1).start()
        lb0 = jnp.where(k >= (DW_CHUNKS + 1) // 2, c0 - DW_CHUNKS, c0) * CHUNK
        _weight_mul(dwloc, lb0, rows0, didx0, CHUNK // L)
        scatter_start(rows0, didx0, ssem0)

        @pl.when(k == DW_CHUNKS // 2)
        def _():
            pltpu.sync_copy(dw_hbm.at[pl.ds(base_e + DW_HALF, DW_REST)],
                            dwloc.at[pl.ds(0, DW_REST)])

        gather(c1, rows1, gsem1).wait()
        scatter_wait(rows0, didx0, ssem0)

        @pl.when(k <= N_FULL // 2 - 2)
        def _():
            gather(c1 + 1, rows0, gsem0).start()
        lb1 = jnp.where(k >= DW_CHUNKS // 2, c1 - DW_CHUNKS, c1) * CHUNK
        _weight_mul(dwloc, lb1, rows1, didx1, CHUNK // L)
        scatter_start(rows1, didx1, ssem1)
        return 0

    lax.fori_loop(0, N_FULL // 2, pipe_body, 0)

    pltpu.async_copy(x_hbm.at[sloc.at[pl.ds(N_FULL * CHUNK, TAIL_E)]],
                     trows, gsem0)
    pltpu.make_async_copy(x_hbm.at[sloc.at[pl.ds(N_FULL * CHUNK, TAIL_E)]],
                          trows, gsem0).wait()
    _weight_mul(dwloc, N_FULL * CHUNK - DW_HALF, trows, dtail, TAIL_E // L)
    pltpu.async_copy(trows, acc.at[dtail], ssem0, add=True)
    scatter_wait(rows1, didx1, ssem1)
    pltpu.make_async_copy(trows, acc.at[dtail], ssem0).wait()
    plsc.subcore_barrier()

    sl = pl.ds(slab0, ROWS_PER_TILE)
    pltpu.sync_copy(acc.at[sl], out_hbm.at[cid, sl])

    @pl.when(sid == NS - 1)
    def _flush_tail():
        tl = pl.ds(NS * ROWS_PER_TILE, TAIL_ROWS)
        pltpu.sync_copy(acc.at[tl], out_hbm.at[cid, tl])


_spmm = functools.partial(
    pl.kernel,
    out_type=jax.ShapeDtypeStruct((NC, N_NODES, D), jnp.float32),
    mesh=plsc.VectorSubcoreMesh(core_axis_name="c", subcore_axis_name="s"),
    compiler_params=pltpu.CompilerParams(needs_layout_passes=False),
    scratch_types=[
        pltpu.VMEM((E_PER_W,), jnp.int32),
        pltpu.VMEM((DW_REST,), jnp.int32),
        pltpu.VMEM((CHUNK, D), jnp.float32),
        pltpu.VMEM((CHUNK, D), jnp.float32),
        pltpu.VMEM((TAIL_E, D), jnp.float32),
        pltpu.VMEM((CHUNK,), jnp.int32),
        pltpu.VMEM((CHUNK,), jnp.int32),
        pltpu.VMEM((TAIL_E,), jnp.int32),
        pltpu.VMEM_SHARED((N_NODES, D), jnp.float32),
        pltpu.SemaphoreType.DMA,
        pltpu.SemaphoreType.DMA,
        pltpu.SemaphoreType.DMA,
        pltpu.SemaphoreType.DMA,
        pltpu.SemaphoreType.DMA,
    ],
)(_spmm_body)


def _mm_body(p_ref, w_ref, b_ref, o_ref):
    s = p_ref[0] + p_ref[1]
    o_ref[...] = (
        jnp.dot(s, w_ref[...], preferred_element_type=jnp.float32)
        + b_ref[...]
    )


M_BLK = 1000


def _fused_matmul(partials, W, b):
    return pl.pallas_call(
        _mm_body,
        grid=(N_NODES // M_BLK,),
        in_specs=[
            pl.BlockSpec((NC, M_BLK, D), lambda i: (0, i, 0)),
            pl.BlockSpec((D, D), lambda i: (0, 0)),
            pl.BlockSpec((1, D), lambda i: (0, 0)),
        ],
        out_specs=pl.BlockSpec((M_BLK, D), lambda i: (i, 0)),
        out_shape=jax.ShapeDtypeStruct((N_NODES, D), jnp.float32),
    )(partials, W, b.reshape(1, D))


def kernel(x, edge_index, edge_weight, W, b):
    ei = edge_index.astype(jnp.int32)
    wbits = lax.bitcast_convert_type(
        edge_weight.astype(jnp.bfloat16), jnp.uint16).astype(jnp.int32)
    dw = (wbits << 16) | ei[0]
    partials = _spmm(x, ei[1], dw)
    return _fused_matmul(partials, W, b)
